# baseline (speedup 1.0000x reference)
import functools

import jax
import jax.numpy as jnp
from jax import lax
from jax.experimental import pallas as pl
from jax.experimental.pallas import tpu as pltpu
from jax.experimental.pallas import tpu_sc as plsc

D_MODEL = 64
CHUNK = 128
LANES = 16
SCALE = 8.0


def _make_sc_gather(n_workers: int, steps: int):
    mesh = plsc.VectorSubcoreMesh(core_axis_name="c", subcore_axis_name="s")

    @functools.partial(
        pl.kernel,
        mesh=mesh,
        out_type=jax.ShapeDtypeStruct((n_workers, steps * CHUNK, D_MODEL),
                                      jnp.float32),
        scratch_types=[
            pltpu.VMEM((steps, CHUNK), jnp.int32),
            pltpu.VMEM((CHUNK, D_MODEL), jnp.float32),
            pltpu.SemaphoreType.DMA,
        ],
        compiler_params=pltpu.CompilerParams(use_tc_tiling_on_sc=False),
    )
    def k(lut_hbm, idx_hbm, out_hbm, idx_v, rows_v, gsem):
        wid = lax.axis_index("s") * 2 + lax.axis_index("c")
        pltpu.sync_copy(idx_hbm.at[wid], idx_v)

        def step_body(j, _):
            pltpu.make_async_copy(
                lut_hbm.at[idx_v.at[j]], rows_v, gsem).start()
            pltpu.make_async_copy(
                lut_hbm.at[idx_v.at[j]], rows_v, gsem).wait()

            def row_body(r, _):
                for c in range(D_MODEL // LANES):
                    rows_v[r, pl.ds(c * LANES, LANES)] = (
                        rows_v[r, pl.ds(c * LANES, LANES)] * SCALE)
                return 0

            lax.fori_loop(0, CHUNK, row_body, 0)
            pltpu.sync_copy(rows_v, out_hbm.at[wid, pl.ds(j * CHUNK, CHUNK)])
            return 0

        lax.fori_loop(0, steps, step_body, 0)

    return k


def kernel(x, lut):
    b, s = x.shape
    n_workers = 32
    total = b * s
    per_worker = total // n_workers
    steps = per_worker // CHUNK
    idx = x.reshape(n_workers, steps, CHUNK).astype(jnp.int32)
    out = _make_sc_gather(n_workers, steps)(lut, idx)
    return out.reshape(b, s, D_MODEL)


# speedup vs baseline: 1.2073x; 1.2073x over previous
import functools

import jax
import jax.numpy as jnp
from jax import lax
from jax.experimental import pallas as pl
from jax.experimental.pallas import tpu as pltpu
from jax.experimental.pallas import tpu_sc as plsc

D_MODEL = 64
CHUNK = 128
LANES = 16
SCALE = 8.0
NBUF = 4


def _make_sc_gather(n_workers: int, steps: int):
    mesh = plsc.VectorSubcoreMesh(core_axis_name="c", subcore_axis_name="s")

    @functools.partial(
        pl.kernel,
        mesh=mesh,
        out_type=jax.ShapeDtypeStruct((n_workers, steps * CHUNK, D_MODEL),
                                      jnp.float32),
        scratch_types=[
            pltpu.VMEM((steps, CHUNK), jnp.int32),
            [pltpu.VMEM((CHUNK, D_MODEL), jnp.float32)] * NBUF,
            [pltpu.VMEM((CHUNK, D_MODEL), jnp.float32)] * NBUF,
            [pltpu.SemaphoreType.DMA] * NBUF,
            [pltpu.SemaphoreType.DMA] * NBUF,
        ],
        compiler_params=pltpu.CompilerParams(use_tc_tiling_on_sc=False),
    )
    def k(lut_hbm, idx_hbm, out_hbm, idx_v, inb, outb, gsem, osem):
        wid = lax.axis_index("s") * 2 + lax.axis_index("c")
        pltpu.sync_copy(idx_hbm.at[wid], idx_v)

        for b in range(NBUF):
            pltpu.make_async_copy(
                lut_hbm.at[idx_v.at[b]], inb[b], gsem[b]).start()

        def group_body(g, _):
            for b in range(NBUF):
                step = g * NBUF + b
                @pl.when(g > 0)
                def _wait_store():
                    pltpu.make_async_copy(
                        outb[b],
                        out_hbm.at[wid, pl.ds((step - NBUF) * CHUNK, CHUNK)],
                        osem[b]).wait()

                pltpu.make_async_copy(
                    lut_hbm.at[idx_v.at[step]], inb[b], gsem[b]).wait()

                def row_body(r, _, b=b):
                    for c in range(D_MODEL // LANES):
                        outb[b][r, pl.ds(c * LANES, LANES)] = (
                            inb[b][r, pl.ds(c * LANES, LANES)] * SCALE)
                    return 0

                lax.fori_loop(0, CHUNK, row_body, 0)

                @pl.when(step + NBUF < steps)
                def _next_gather():
                    pltpu.make_async_copy(
                        lut_hbm.at[idx_v.at[step + NBUF]], inb[b],
                        gsem[b]).start()

                pltpu.make_async_copy(
                    outb[b], out_hbm.at[wid, pl.ds(step * CHUNK, CHUNK)],
                    osem[b]).start()
            return 0

        lax.fori_loop(0, steps // NBUF, group_body, 0)

        for b in range(NBUF):
            pltpu.make_async_copy(
                outb[b],
                out_hbm.at[wid, pl.ds((steps - NBUF + b) * CHUNK, CHUNK)],
                osem[b]).wait()

    return k


def kernel(x, lut):
    b, s = x.shape
    n_workers = 32
    total = b * s
    per_worker = total // n_workers
    steps = per_worker // CHUNK
    idx = x.reshape(n_workers, steps, CHUNK).astype(jnp.int32)
    out = _make_sc_gather(n_workers, steps)(lut, idx)
    return out.reshape(b, s, D_MODEL)
